# baseline (speedup 1.0000x reference)
import jax
import jax.numpy as jnp
from jax import lax
from jax.experimental import pallas as pl
from jax.experimental.pallas import tpu as pltpu
from jax.experimental.pallas import tpu_sc as plsc

_N = 10000
_E = 320000
_D = 128
_HD = 64
_G = 64

_NC, _NS = 2, 16
_NW = _NC * _NS
_KH = 80
_EPW = _E // _NW
_NCHH = _EPW // _KH
_K = 50
_EPS = _E // _NS
_NCH = _EPS // _K
_NBUF = 8
_NGRP = _NCH // _NBUF
_RPS = _N // _NS
_ZR = 125

_RB = 400
_NB = _N // _RB

_SC_PARAMS = pltpu.CompilerParams(use_tc_tiling_on_sc=False)


def _sc_mesh():
    return plsc.VectorSubcoreMesh(
        core_axis_name="c", subcore_axis_name="s",
        num_cores=_NC, num_subcores=_NS)


def _hist_body(dst_hbm, out_hbm, idx_v, ones_v, zro_v, hist_sh, semh):
    cid = lax.axis_index("c")
    sid = lax.axis_index("s")
    wid = cid * _NS + sid

    hidx = pltpu.async_copy(dst_hbm.at[wid], idx_v, semh)

    @pl.loop(0, _KH)
    def _(r):
        ones_v[r, :] = jnp.full((16,), 1.0, jnp.float32)

    @pl.loop(0, _RPS)
    def _(r):
        zro_v[r, :] = jnp.zeros((16,), jnp.float32)

    pltpu.sync_copy(zro_v, hist_sh.at[pl.ds(sid * _RPS, _RPS)])
    hidx.wait()
    plsc.subcore_barrier()

    @pl.loop(0, _NCHH)
    def _(c):
        pltpu.async_copy(ones_v, hist_sh.at[idx_v.at[c]], semh, add=True)

    @pl.loop(0, _NCHH)
    def _(c):
        pltpu.make_async_copy(out_hbm.at[cid, sid].at[pl.ds(0, _KH)],
                              ones_v, semh).wait()

    plsc.subcore_barrier()
    pltpu.sync_copy(hist_sh.at[pl.ds(sid * _RPS, _RPS)],
                    out_hbm.at[cid, sid])


def _sc_hist(dst3):
    k = pl.kernel(
        _hist_body,
        out_type=jax.ShapeDtypeStruct((_NC, _NS, _RPS, 16), jnp.float32),
        mesh=_sc_mesh(),
        scratch_types=[
            pltpu.VMEM((_NCHH, _KH), jnp.int32),
            pltpu.VMEM((_KH, 16), jnp.float32),
            pltpu.VMEM((_RPS, 16), jnp.float32),
            pltpu.VMEM_SHARED((_N, 16), jnp.float32),
            pltpu.SemaphoreType.DMA,
        ],
        compiler_params=_SC_PARAMS,
    )
    return k(dst3)


def _agg_body(p_hbm, src_hbm, dst_hbm, out_hbm, srcv, dstv,
              r0, r1, r2, r3, r4, r5, r6, r7, zb, acc_sh,
              s0, s1, s2, s3, s4, s5, s6, s7):
    cid = lax.axis_index("c")
    sid = lax.axis_index("s")
    rows = (r0, r1, r2, r3, r4, r5, r6, r7)
    sems = (s0, s1, s2, s3, s4, s5, s6, s7)

    hsrc = pltpu.async_copy(src_hbm.at[sid], srcv, s0)
    hdst = pltpu.async_copy(dst_hbm.at[sid], dstv, s1)

    @pl.loop(0, _ZR)
    def _(r):
        @pl.loop(0, _HD // 16)
        def _(c16):
            zb[r, pl.ds(c16 * 16, 16)] = jnp.zeros((16,), jnp.float32)

    for j in range(_RPS // _ZR):
        pltpu.sync_copy(zb, acc_sh.at[pl.ds(sid * _RPS + j * _ZR, _ZR)])

    hsrc.wait()
    hdst.wait()
    plsc.subcore_barrier()

    phalf = p_hbm.at[cid]

    def _fire_gather(c, j):
        pltpu.async_copy(phalf.at[srcv.at[c]], rows[j], sems[j])

    def _fire_scatter(c, j):
        pltpu.async_copy(rows[j], acc_sh.at[dstv.at[c]], sems[j], add=True)

    def _wait(j):
        pltpu.make_async_copy(phalf.at[pl.ds(0, _K)], rows[j], sems[j]).wait()

    for j in range(_NBUF):
        _fire_gather(j, j)

    @pl.loop(0, _NGRP - 1)
    def _(g):
        c0 = g * _NBUF
        for j in range(_NBUF):
            _wait(j)
            _fire_scatter(c0 + j, j)
        for j in range(_NBUF):
            _wait(j)
            _fire_gather(c0 + _NBUF + j, j)

    cl = (_NGRP - 1) * _NBUF
    for j in range(_NBUF):
        _wait(j)
        _fire_scatter(cl + j, j)
    for j in range(_NBUF):
        _wait(j)

    plsc.subcore_barrier()
    pltpu.sync_copy(acc_sh.at[pl.ds(sid * _RPS, _RPS)],
                    out_hbm.at[cid, sid])


def _sc_agg(p2, src3, dst3):
    k = pl.kernel(
        _agg_body,
        out_type=jax.ShapeDtypeStruct((_NC, _NS, _RPS, _HD), jnp.float32),
        mesh=_sc_mesh(),
        scratch_types=[
            pltpu.VMEM((_NCH, _K), jnp.int32),
            pltpu.VMEM((_NCH, _K), jnp.int32),
            pltpu.VMEM((_K, _HD), jnp.float32),
            pltpu.VMEM((_K, _HD), jnp.float32),
            pltpu.VMEM((_K, _HD), jnp.float32),
            pltpu.VMEM((_K, _HD), jnp.float32),
            pltpu.VMEM((_K, _HD), jnp.float32),
            pltpu.VMEM((_K, _HD), jnp.float32),
            pltpu.VMEM((_K, _HD), jnp.float32),
            pltpu.VMEM((_K, _HD), jnp.float32),
            pltpu.VMEM((_ZR, _HD), jnp.float32),
            pltpu.VMEM_SHARED((_N, _HD), jnp.float32),
            pltpu.SemaphoreType.DMA,
            pltpu.SemaphoreType.DMA,
            pltpu.SemaphoreType.DMA,
            pltpu.SemaphoreType.DMA,
            pltpu.SemaphoreType.DMA,
            pltpu.SemaphoreType.DMA,
            pltpu.SemaphoreType.DMA,
            pltpu.SemaphoreType.DMA,
        ],
        compiler_params=_SC_PARAMS,
    )
    return k(p2, src3, dst3)


def _dis_block(hist_ref, i):
    d0 = hist_ref[0, pl.ds(i * _RB, _RB), 0:1]
    d1 = hist_ref[1, pl.ds(i * _RB, _RB), 0:1]
    return lax.rsqrt(d0 + d1 + 1.0)


def _dot(a, b):
    return jnp.dot(a, b, preferred_element_type=jnp.float32,
                   precision=lax.Precision.HIGHEST)


def _split_out(o_ref, p):
    o_ref[0] = p[:, :_HD]
    o_ref[1] = p[:, _HD:]


def _p1_body(hist_ref, x_ref, w_ref, o_ref):
    i = pl.program_id(0)
    dis = _dis_block(hist_ref, i)
    _split_out(o_ref, dis * _dot(x_ref[...], w_ref[...]))


def _tc_p1(hist, x, W1):
    return pl.pallas_call(
        _p1_body,
        grid=(_NB,),
        in_specs=[
            pl.BlockSpec((_NC, _N, 16), lambda i: (0, 0, 0)),
            pl.BlockSpec((_RB, _D), lambda i: (i, 0)),
            pl.BlockSpec((_D, _D), lambda i: (0, 0)),
        ],
        out_specs=pl.BlockSpec((_NC, _RB, _HD), lambda i: (0, i, 0)),
        out_shape=jax.ShapeDtypeStruct((_NC, _N, _HD), jnp.float32),
    )(hist, x, W1)


def _relu_h(hist_ref, agg_ref, p_ref, b_ref, i):
    dis = _dis_block(hist_ref, i)
    s = jnp.concatenate([agg_ref[0] + p_ref[0], agg_ref[1] + p_ref[1]],
                        axis=-1)
    return jnp.maximum(dis * s + b_ref[...], 0.0)


def _mid_body(hist_ref, agg_ref, p_ref, b_ref, w_ref, o_ref):
    i = pl.program_id(0)
    h = _relu_h(hist_ref, agg_ref, p_ref, b_ref, i)
    dis = _dis_block(hist_ref, i)
    _split_out(o_ref, dis * _dot(h, w_ref[...]))


def _tc_mid(hist, agg, p, b, W):
    return pl.pallas_call(
        _mid_body,
        grid=(_NB,),
        in_specs=[
            pl.BlockSpec((_NC, _N, 16), lambda i: (0, 0, 0)),
            pl.BlockSpec((_NC, _RB, _HD), lambda i: (0, i, 0)),
            pl.BlockSpec((_NC, _RB, _HD), lambda i: (0, i, 0)),
            pl.BlockSpec((1, _D), lambda i: (0, 0)),
            pl.BlockSpec((_D, _D), lambda i: (0, 0)),
        ],
        out_specs=pl.BlockSpec((_NC, _RB, _HD), lambda i: (0, i, 0)),
        out_shape=jax.ShapeDtypeStruct((_NC, _N, _HD), jnp.float32),
    )(hist, agg, p, b, W)


def _pool_body(hist_ref, agg_ref, p_ref, b_ref, batch_ref, o_ref, acc, cnt):
    i = pl.program_id(0)

    @pl.when(i == 0)
    def _():
        acc[...] = jnp.zeros_like(acc)
        cnt[...] = jnp.zeros_like(cnt)

    h = _relu_h(hist_ref, agg_ref, p_ref, b_ref, i)
    bb = batch_ref[i, :]
    oh = (lax.broadcasted_iota(jnp.int32, (_G, _RB), 0) == bb[None, :])
    oh = oh.astype(jnp.float32)
    acc[...] += _dot(oh, h)
    cnt[...] += _dot(oh, jnp.ones((_RB, _D), jnp.float32))

    @pl.when(i == _NB - 1)
    def _():
        o_ref[...] = acc[...] / jnp.maximum(cnt[...], 1.0)


def _tc_pool(hist, agg, p, b, batch2):
    return pl.pallas_call(
        _pool_body,
        grid=(_NB,),
        in_specs=[
            pl.BlockSpec((_NC, _N, 16), lambda i: (0, 0, 0)),
            pl.BlockSpec((_NC, _RB, _HD), lambda i: (0, i, 0)),
            pl.BlockSpec((_NC, _RB, _HD), lambda i: (0, i, 0)),
            pl.BlockSpec((1, _D), lambda i: (0, 0)),
            pl.BlockSpec((_NB, _RB), lambda i: (0, 0)),
        ],
        out_specs=pl.BlockSpec((_G, _D), lambda i: (0, 0)),
        out_shape=jax.ShapeDtypeStruct((_G, _D), jnp.float32),
        scratch_shapes=[
            pltpu.VMEM((_G, _D), jnp.float32),
            pltpu.VMEM((_G, _D), jnp.float32),
        ],
        compiler_params=pltpu.CompilerParams(
            dimension_semantics=("arbitrary",)),
    )(hist, agg, p, b, batch2)


def kernel(x, edge_index, batch, W1, b1, W2, b2, W3, b3):
    dsth = edge_index[1].reshape(_NW, _NCHH, _KH)
    src3 = edge_index[0].reshape(_NS, _NCH, _K)
    dst3 = edge_index[1].reshape(_NS, _NCH, _K)
    batch2 = batch.reshape(_NB, _RB)

    hist = _sc_hist(dsth).reshape(_NC, _N, 16)
    p1 = _tc_p1(hist, x, W1)
    a1 = _sc_agg(p1, src3, dst3).reshape(_NC, _N, _HD)
    p2 = _tc_mid(hist, a1, p1, b1.reshape(1, _D), W2)
    a2 = _sc_agg(p2, src3, dst3).reshape(_NC, _N, _HD)
    p3 = _tc_mid(hist, a2, p2, b2.reshape(1, _D), W3)
    a3 = _sc_agg(p3, src3, dst3).reshape(_NC, _N, _HD)
    return _tc_pool(hist, a3, p3, b3.reshape(1, _D), batch2)


# speedup vs baseline: 1.0339x; 1.0339x over previous
import jax
import jax.numpy as jnp
from jax import lax
from jax.experimental import pallas as pl
from jax.experimental.pallas import tpu as pltpu
from jax.experimental.pallas import tpu_sc as plsc

_N = 10000
_E = 320000
_D = 128
_HD = 64
_G = 64

_NC, _NS = 2, 16
_NW = _NC * _NS
_KH = 80
_EPW = _E // _NW
_NCHH = _EPW // _KH
_K = 125
_EPS = _E // _NS
_NCH = _EPS // _K
_NBUF = 5
_NGRP = _NCH // _NBUF
_RPS = _N // _NS
_ZR = 125

_RB = 400
_NB = _N // _RB

_SC_PARAMS = pltpu.CompilerParams(use_tc_tiling_on_sc=False)


def _sc_mesh():
    return plsc.VectorSubcoreMesh(
        core_axis_name="c", subcore_axis_name="s",
        num_cores=_NC, num_subcores=_NS)


def _hist_body(dst_hbm, out_hbm, idx_v, ones_v, zro_v, hist_sh, semh):
    cid = lax.axis_index("c")
    sid = lax.axis_index("s")
    wid = cid * _NS + sid

    hidx = pltpu.async_copy(dst_hbm.at[wid], idx_v, semh)

    @pl.loop(0, _KH)
    def _(r):
        ones_v[r, :] = jnp.full((16,), 1.0, jnp.float32)

    @pl.loop(0, _RPS)
    def _(r):
        zro_v[r, :] = jnp.zeros((16,), jnp.float32)

    pltpu.sync_copy(zro_v, hist_sh.at[pl.ds(sid * _RPS, _RPS)])
    hidx.wait()
    plsc.subcore_barrier()

    @pl.loop(0, _NCHH)
    def _(c):
        pltpu.async_copy(ones_v, hist_sh.at[idx_v.at[c]], semh, add=True)

    @pl.loop(0, _NCHH)
    def _(c):
        pltpu.make_async_copy(out_hbm.at[cid, sid].at[pl.ds(0, _KH)],
                              ones_v, semh).wait()

    plsc.subcore_barrier()
    pltpu.sync_copy(hist_sh.at[pl.ds(sid * _RPS, _RPS)],
                    out_hbm.at[cid, sid])


def _sc_hist(dst3):
    k = pl.kernel(
        _hist_body,
        out_type=jax.ShapeDtypeStruct((_NC, _NS, _RPS, 16), jnp.float32),
        mesh=_sc_mesh(),
        scratch_types=[
            pltpu.VMEM((_NCHH, _KH), jnp.int32),
            pltpu.VMEM((_KH, 16), jnp.float32),
            pltpu.VMEM((_RPS, 16), jnp.float32),
            pltpu.VMEM_SHARED((_N, 16), jnp.float32),
            pltpu.SemaphoreType.DMA,
        ],
        compiler_params=_SC_PARAMS,
    )
    return k(dst3)


def _agg_body(p_hbm, src_hbm, dst_hbm, out_hbm, srcv, dstv,
              r0, r1, r2, r3, r4, zb, acc_sh, s0, s1, s2, s3, s4):
    cid = lax.axis_index("c")
    sid = lax.axis_index("s")
    rows = (r0, r1, r2, r3, r4)
    sems = (s0, s1, s2, s3, s4)

    hsrc = pltpu.async_copy(src_hbm.at[sid], srcv, s0)
    hdst = pltpu.async_copy(dst_hbm.at[sid], dstv, s1)

    @pl.loop(0, _ZR)
    def _(r):
        @pl.loop(0, _HD // 16)
        def _(c16):
            zb[r, pl.ds(c16 * 16, 16)] = jnp.zeros((16,), jnp.float32)

    for j in range(_RPS // _ZR):
        pltpu.sync_copy(zb, acc_sh.at[pl.ds(sid * _RPS + j * _ZR, _ZR)])

    hsrc.wait()
    hdst.wait()
    plsc.subcore_barrier()

    phalf = p_hbm.at[cid]

    def _fire_gather(c, j):
        pltpu.async_copy(phalf.at[srcv.at[c]], rows[j], sems[j])

    def _fire_scatter(c, j):
        pltpu.async_copy(rows[j], acc_sh.at[dstv.at[c]], sems[j], add=True)

    def _wait(j):
        pltpu.make_async_copy(phalf.at[pl.ds(0, _K)], rows[j], sems[j]).wait()

    for j in range(_NBUF):
        _fire_gather(j, j)

    @pl.loop(0, _NGRP - 1)
    def _(g):
        c0 = g * _NBUF
        for j in range(_NBUF):
            _wait(j)
            _fire_scatter(c0 + j, j)
        for j in range(_NBUF):
            _wait(j)
            _fire_gather(c0 + _NBUF + j, j)

    cl = (_NGRP - 1) * _NBUF
    for j in range(_NBUF):
        _wait(j)
        _fire_scatter(cl + j, j)
    for j in range(_NBUF):
        _wait(j)

    plsc.subcore_barrier()
    pltpu.sync_copy(acc_sh.at[pl.ds(sid * _RPS, _RPS)],
                    out_hbm.at[cid, sid])


def _sc_agg(p2, src3, dst3):
    k = pl.kernel(
        _agg_body,
        out_type=jax.ShapeDtypeStruct((_NC, _NS, _RPS, _HD), jnp.float32),
        mesh=_sc_mesh(),
        scratch_types=[
            pltpu.VMEM((_NCH, _K), jnp.int32),
            pltpu.VMEM((_NCH, _K), jnp.int32),
            pltpu.VMEM((_K, _HD), jnp.float32),
            pltpu.VMEM((_K, _HD), jnp.float32),
            pltpu.VMEM((_K, _HD), jnp.float32),
            pltpu.VMEM((_K, _HD), jnp.float32),
            pltpu.VMEM((_K, _HD), jnp.float32),
            pltpu.VMEM((_ZR, _HD), jnp.float32),
            pltpu.VMEM_SHARED((_N, _HD), jnp.float32),
            pltpu.SemaphoreType.DMA,
            pltpu.SemaphoreType.DMA,
            pltpu.SemaphoreType.DMA,
            pltpu.SemaphoreType.DMA,
            pltpu.SemaphoreType.DMA,
        ],
        compiler_params=_SC_PARAMS,
    )
    return k(p2, src3, dst3)


def _dis_block(hist_ref, i):
    d0 = hist_ref[0, pl.ds(i * _RB, _RB), 0:1]
    d1 = hist_ref[1, pl.ds(i * _RB, _RB), 0:1]
    return lax.rsqrt(d0 + d1 + 1.0)


def _dot(a, b):
    return jnp.dot(a, b, preferred_element_type=jnp.float32,
                   precision=lax.Precision.HIGHEST)


def _split_out(o_ref, p):
    o_ref[0] = p[:, :_HD]
    o_ref[1] = p[:, _HD:]


def _p1_body(hist_ref, x_ref, w_ref, o_ref):
    i = pl.program_id(0)
    dis = _dis_block(hist_ref, i)
    _split_out(o_ref, dis * _dot(x_ref[...], w_ref[...]))


def _tc_p1(hist, x, W1):
    return pl.pallas_call(
        _p1_body,
        grid=(_NB,),
        in_specs=[
            pl.BlockSpec((_NC, _N, 16), lambda i: (0, 0, 0)),
            pl.BlockSpec((_RB, _D), lambda i: (i, 0)),
            pl.BlockSpec((_D, _D), lambda i: (0, 0)),
        ],
        out_specs=pl.BlockSpec((_NC, _RB, _HD), lambda i: (0, i, 0)),
        out_shape=jax.ShapeDtypeStruct((_NC, _N, _HD), jnp.float32),
    )(hist, x, W1)


def _relu_h(hist_ref, agg_ref, p_ref, b_ref, i):
    dis = _dis_block(hist_ref, i)
    s = jnp.concatenate([agg_ref[0] + p_ref[0], agg_ref[1] + p_ref[1]],
                        axis=-1)
    return jnp.maximum(dis * s + b_ref[...], 0.0)


def _mid_body(hist_ref, agg_ref, p_ref, b_ref, w_ref, o_ref):
    i = pl.program_id(0)
    h = _relu_h(hist_ref, agg_ref, p_ref, b_ref, i)
    dis = _dis_block(hist_ref, i)
    _split_out(o_ref, dis * _dot(h, w_ref[...]))


def _tc_mid(hist, agg, p, b, W):
    return pl.pallas_call(
        _mid_body,
        grid=(_NB,),
        in_specs=[
            pl.BlockSpec((_NC, _N, 16), lambda i: (0, 0, 0)),
            pl.BlockSpec((_NC, _RB, _HD), lambda i: (0, i, 0)),
            pl.BlockSpec((_NC, _RB, _HD), lambda i: (0, i, 0)),
            pl.BlockSpec((1, _D), lambda i: (0, 0)),
            pl.BlockSpec((_D, _D), lambda i: (0, 0)),
        ],
        out_specs=pl.BlockSpec((_NC, _RB, _HD), lambda i: (0, i, 0)),
        out_shape=jax.ShapeDtypeStruct((_NC, _N, _HD), jnp.float32),
    )(hist, agg, p, b, W)


def _pool_body(hist_ref, agg_ref, p_ref, b_ref, batch_ref, o_ref, acc, cnt):
    i = pl.program_id(0)

    @pl.when(i == 0)
    def _():
        acc[...] = jnp.zeros_like(acc)
        cnt[...] = jnp.zeros_like(cnt)

    h = _relu_h(hist_ref, agg_ref, p_ref, b_ref, i)
    bb = batch_ref[i, :]
    oh = (lax.broadcasted_iota(jnp.int32, (_G, _RB), 0) == bb[None, :])
    oh = oh.astype(jnp.float32)
    acc[...] += _dot(oh, h)
    cnt[...] += _dot(oh, jnp.ones((_RB, _D), jnp.float32))

    @pl.when(i == _NB - 1)
    def _():
        o_ref[...] = acc[...] / jnp.maximum(cnt[...], 1.0)


def _tc_pool(hist, agg, p, b, batch2):
    return pl.pallas_call(
        _pool_body,
        grid=(_NB,),
        in_specs=[
            pl.BlockSpec((_NC, _N, 16), lambda i: (0, 0, 0)),
            pl.BlockSpec((_NC, _RB, _HD), lambda i: (0, i, 0)),
            pl.BlockSpec((_NC, _RB, _HD), lambda i: (0, i, 0)),
            pl.BlockSpec((1, _D), lambda i: (0, 0)),
            pl.BlockSpec((_NB, _RB), lambda i: (0, 0)),
        ],
        out_specs=pl.BlockSpec((_G, _D), lambda i: (0, 0)),
        out_shape=jax.ShapeDtypeStruct((_G, _D), jnp.float32),
        scratch_shapes=[
            pltpu.VMEM((_G, _D), jnp.float32),
            pltpu.VMEM((_G, _D), jnp.float32),
        ],
        compiler_params=pltpu.CompilerParams(
            dimension_semantics=("arbitrary",)),
    )(hist, agg, p, b, batch2)


def kernel(x, edge_index, batch, W1, b1, W2, b2, W3, b3):
    dsth = edge_index[1].reshape(_NW, _NCHH, _KH)
    src3 = edge_index[0].reshape(_NS, _NCH, _K)
    dst3 = edge_index[1].reshape(_NS, _NCH, _K)
    batch2 = batch.reshape(_NB, _RB)

    hist = _sc_hist(dsth).reshape(_NC, _N, 16)
    p1 = _tc_p1(hist, x, W1)
    a1 = _sc_agg(p1, src3, dst3).reshape(_NC, _N, _HD)
    p2 = _tc_mid(hist, a1, p1, b1.reshape(1, _D), W2)
    a2 = _sc_agg(p2, src3, dst3).reshape(_NC, _N, _HD)
    p3 = _tc_mid(hist, a2, p2, b2.reshape(1, _D), W3)
    a3 = _sc_agg(p3, src3, dst3).reshape(_NC, _N, _HD)
    return _tc_pool(hist, a3, p3, b3.reshape(1, _D), batch2)
